# initial kernel scaffold (unmeasured)
import jax
import jax.numpy as jnp
from jax import lax
from jax.experimental import pallas as pl
from jax.experimental.pallas import tpu as pltpu

N_DEV = 8
M_PER = 512
K = 4096
N_TOT = 8192
N_PER = 1024


def kernel(x, w_mat):
    def body(x_ref, w_ref, out_ref, w_bufs, y_buf, q_buf, data_buf, amax_buf,
             w_sems, amax_send_sems, amax_recv_sems,
             data_send_sems, data_recv_sems):
        me = lax.axis_index("i")

        barrier_sem = pltpu.get_barrier_semaphore()
        for k in range(1, N_DEV):
            peer = lax.rem(me + k, N_DEV)
            pl.semaphore_signal(barrier_sem, inc=1, device_id=(peer,),
                                device_id_type=pl.DeviceIdType.MESH)
        pl.semaphore_wait(barrier_sem, N_DEV - 1)

        def w_copy(j, slot):
            return pltpu.make_async_copy(
                w_ref.at[:, pl.ds(j * N_PER, N_PER)],
                w_bufs.at[slot],
                w_sems.at[slot],
            )

        w_copy(0, 0).start()
        amax = jnp.float32(0.0)
        for j in range(N_DEV):
            slot = j % 2
            if j + 1 < N_DEV:
                w_copy(j + 1, (j + 1) % 2).start()
            w_copy(j, slot).wait()
            yj = jnp.dot(x_ref[...], w_bufs[slot],
                         preferred_element_type=jnp.float32)
            y_buf[:, pl.ds(j * N_PER, N_PER)] = yj
            amax = jnp.maximum(amax, jnp.max(jnp.abs(yj)))

        amax_buf[pl.ds(me, 1), :] = jnp.full((1, 128), amax, jnp.float32)
        amax_sends = []
        for k in range(1, N_DEV):
            peer = lax.rem(me + k, N_DEV)
            r = pltpu.make_async_remote_copy(
                src_ref=amax_buf.at[pl.ds(me, 1)],
                dst_ref=amax_buf.at[pl.ds(me, 1)],
                send_sem=amax_send_sems.at[peer],
                recv_sem=amax_recv_sems.at[me],
                device_id=(peer,),
                device_id_type=pl.DeviceIdType.MESH,
            )
            r.start()
            amax_sends.append(r)
        for k in range(1, N_DEV):
            peer = lax.rem(me + k, N_DEV)
            pltpu.make_async_remote_copy(
                src_ref=amax_buf.at[pl.ds(peer, 1)],
                dst_ref=amax_buf.at[pl.ds(peer, 1)],
                send_sem=amax_send_sems.at[peer],
                recv_sem=amax_recv_sems.at[peer],
                device_id=(peer,),
                device_id_type=pl.DeviceIdType.MESH,
            ).wait_recv()
        global_amax = jnp.max(amax_buf[...])
        scale = global_amax / 127.0

        q = jnp.clip(jnp.round(y_buf[...] / scale), -127.0, 127.0)
        q_buf[...] = q.astype(jnp.int8)

        data_buf[pl.ds(me, 1)] = jnp.expand_dims(
            q_buf[:, pl.ds(me * N_PER, N_PER)], 0)

        data_sends = []
        for k in range(1, N_DEV):
            peer = lax.rem(me + k, N_DEV)
            r = pltpu.make_async_remote_copy(
                src_ref=q_buf.at[:, pl.ds(peer * N_PER, N_PER)],
                dst_ref=data_buf.at[me],
                send_sem=data_send_sems.at[peer],
                recv_sem=data_recv_sems.at[me],
                device_id=(peer,),
                device_id_type=pl.DeviceIdType.MESH,
            )
            r.start()
            data_sends.append(r)
        for k in range(1, N_DEV):
            peer = lax.rem(me + k, N_DEV)
            pltpu.make_async_remote_copy(
                src_ref=q_buf.at[:, pl.ds(peer * N_PER, N_PER)],
                dst_ref=data_buf.at[peer],
                send_sem=data_send_sems.at[peer],
                recv_sem=data_recv_sems.at[peer],
                device_id=(peer,),
                device_id_type=pl.DeviceIdType.MESH,
            ).wait_recv()

        for j in range(N_DEV):
            out_ref[pl.ds(j * M_PER, M_PER), :] = (
                data_buf[j].astype(jnp.float32) * scale)

        for r in amax_sends + data_sends:
            r.wait_send()

    return pl.pallas_call(
        body,
        out_shape=jax.ShapeDtypeStruct((N_DEV * M_PER, N_PER), jnp.float32),
        in_specs=[
            pl.BlockSpec(memory_space=pltpu.VMEM),
            pl.BlockSpec(memory_space=pltpu.ANY),
        ],
        out_specs=pl.BlockSpec(memory_space=pltpu.VMEM),
        scratch_shapes=[
            pltpu.VMEM((2, K, N_PER), jnp.float32),
            pltpu.VMEM((M_PER, N_TOT), jnp.float32),
            pltpu.VMEM((M_PER, N_TOT), jnp.int8),
            pltpu.VMEM((N_DEV, M_PER, N_PER), jnp.int8),
            pltpu.VMEM((N_DEV, 128), jnp.float32),
            pltpu.SemaphoreType.DMA((2,)),
            pltpu.SemaphoreType.DMA((N_DEV,)),
            pltpu.SemaphoreType.DMA((N_DEV,)),
            pltpu.SemaphoreType.DMA((N_DEV,)),
            pltpu.SemaphoreType.DMA((N_DEV,)),
        ],
        compiler_params=pltpu.CompilerParams(
            collective_id=0,
            vmem_limit_bytes=128 * 1024 * 1024,
        ),
    )(x, w_mat)


# baseline (device time: 83494 ns/iter reference)
import jax
import jax.numpy as jnp
from jax import lax
from jax.experimental import pallas as pl
from jax.experimental.pallas import tpu as pltpu

N_DEV = 8
M_PER = 512
K = 4096
N_PER = 1024
TILE = 512
N_TILES = 2 * N_DEV


def kernel(x, w_mat):
    def body(x_ref, w_ref, out_ref, w_bufs, y_blk, q_bufs, data_buf,
             scale_send, scale_recv, w_sems,
             data_send_sems, data_recv_sems,
             scale_send_sems, scale_recv_sems):
        me = lax.axis_index("i")

        barrier_sem = pltpu.get_barrier_semaphore()
        for k in range(1, N_DEV):
            peer = lax.rem(me + k, N_DEV)
            pl.semaphore_signal(barrier_sem, inc=1, device_id=(peer,),
                                device_id_type=pl.DeviceIdType.MESH)
        pl.semaphore_wait(barrier_sem, N_DEV - 1)

        def w_copy(t, slot):
            blk = lax.rem(me + 1 + t // 2, N_DEV)
            start = blk * N_PER + (t % 2) * TILE
            return pltpu.make_async_copy(
                w_ref.at[:, pl.ds(start, TILE)],
                w_bufs.at[slot],
                w_sems.at[slot],
            )

        w_copy(0, 0).start()
        w_copy(1, 1).start()

        sends = []
        for k in range(1, N_DEV + 1):
            j = lax.rem(me + k, N_DEV)
            for t in range(2):
                t_lin = 2 * (k - 1) + t
                slot = t_lin % 2
                w_copy(t_lin, slot).wait()
                y_blk[:, pl.ds(t * TILE, TILE)] = jnp.dot(
                    x_ref[...], w_bufs[slot],
                    preferred_element_type=jnp.float32)
                if t_lin + 2 < N_TILES:
                    w_copy(t_lin + 2, slot).start()

            yb = y_blk[...]
            amax_b = jnp.maximum(jnp.max(jnp.abs(yb)), jnp.float32(1e-30))
            scale_b = amax_b / 127.0
            q_bufs[k - 1] = jnp.clip(
                jnp.round(yb / scale_b), -127.0, 127.0).astype(jnp.int8)
            scale_send[k - 1, :] = jnp.full((128,), scale_b, jnp.float32)

            if k < N_DEV:
                r = pltpu.make_async_remote_copy(
                    src_ref=q_bufs.at[k - 1],
                    dst_ref=data_buf.at[me],
                    send_sem=data_send_sems.at[k - 1],
                    recv_sem=data_recv_sems.at[me],
                    device_id=(j,),
                    device_id_type=pl.DeviceIdType.MESH,
                )
                r.start()
                sends.append(r)
                rs = pltpu.make_async_remote_copy(
                    src_ref=scale_send.at[pl.ds(k - 1, 1)],
                    dst_ref=scale_recv.at[pl.ds(me, 1)],
                    send_sem=scale_send_sems.at[k - 1],
                    recv_sem=scale_recv_sems.at[me],
                    device_id=(j,),
                    device_id_type=pl.DeviceIdType.MESH,
                )
                rs.start()
                sends.append(rs)
            else:
                data_buf[pl.ds(me, 1)] = jnp.expand_dims(q_bufs[N_DEV - 1], 0)
                scale_recv[pl.ds(me, 1), :] = jnp.full(
                    (1, 128), scale_b, jnp.float32)

        for k in range(1, N_DEV):
            p = lax.rem(me + k, N_DEV)
            pltpu.make_async_remote_copy(
                src_ref=q_bufs.at[0],
                dst_ref=data_buf.at[p],
                send_sem=data_send_sems.at[0],
                recv_sem=data_recv_sems.at[p],
                device_id=(p,),
                device_id_type=pl.DeviceIdType.MESH,
            ).wait_recv()
            pltpu.make_async_remote_copy(
                src_ref=scale_send.at[pl.ds(0, 1)],
                dst_ref=scale_recv.at[pl.ds(p, 1)],
                send_sem=scale_send_sems.at[0],
                recv_sem=scale_recv_sems.at[p],
                device_id=(p,),
                device_id_type=pl.DeviceIdType.MESH,
            ).wait_recv()

        for j in range(N_DEV):
            out_ref[pl.ds(j * M_PER, M_PER), :] = (
                data_buf[j].astype(jnp.float32) * scale_recv[j, 0])

        for r in sends:
            r.wait_send()

    return pl.pallas_call(
        body,
        out_shape=jax.ShapeDtypeStruct((N_DEV * M_PER, N_PER), jnp.float32),
        in_specs=[
            pl.BlockSpec(memory_space=pltpu.VMEM),
            pl.BlockSpec(memory_space=pl.ANY),
        ],
        out_specs=pl.BlockSpec(memory_space=pltpu.VMEM),
        scratch_shapes=[
            pltpu.VMEM((2, K, TILE), jnp.float32),
            pltpu.VMEM((M_PER, N_PER), jnp.float32),
            pltpu.VMEM((N_DEV, M_PER, N_PER), jnp.int8),
            pltpu.VMEM((N_DEV, M_PER, N_PER), jnp.int8),
            pltpu.VMEM((N_DEV, 128), jnp.float32),
            pltpu.VMEM((N_DEV, 128), jnp.float32),
            pltpu.SemaphoreType.DMA((2,)),
            pltpu.SemaphoreType.DMA((N_DEV,)),
            pltpu.SemaphoreType.DMA((N_DEV,)),
            pltpu.SemaphoreType.DMA((N_DEV,)),
            pltpu.SemaphoreType.DMA((N_DEV,)),
        ],
        compiler_params=pltpu.CompilerParams(
            collective_id=0,
            vmem_limit_bytes=64 * 1024 * 1024,
        ),
    )(x, w_mat)
